# baseline (speedup 1.0000x reference)
import functools

import jax
import jax.numpy as jnp
from jax import lax
from jax.experimental import pallas as pl
from jax.experimental.pallas import tpu as pltpu
from jax.experimental.pallas import tpu_sc as plsc

_EPS = 1e-5

_NC = 2
_NS = 16
_NW = _NC * _NS

_CHUNK = 128

_CHUNK_A = 128
_NBA = 2


def _sc_mesh():
    return plsc.VectorSubcoreMesh(
        core_axis_name="c", subcore_axis_name="s",
        num_cores=_NC, num_subcores=_NS)


def _sc_deg(dst, ones_c, zeros_n):
    n = zeros_n.shape[0]
    e = dst.shape[0]
    epw = e // _NW
    nchunk = epw // _CHUNK
    nb = 6
    ngroup = nchunk // nb
    gtail = nchunk - ngroup * nb
    etail = epw - nchunk * _CHUNK

    isems = [pltpu.SemaphoreType.DMA for _ in range(nb)]
    tail_types = ([pltpu.VMEM((etail,), jnp.int32),
                   pltpu.VMEM((etail,), jnp.float32)] if etail else [])

    @functools.partial(
        pl.kernel,
        out_type=jax.ShapeDtypeStruct((_NC, n), jnp.float32),
        mesh=_sc_mesh(),
        scratch_types=[
            pltpu.VMEM((_CHUNK,), jnp.float32),
            pltpu.VMEM_SHARED((n,), jnp.float32),
            pltpu.SemaphoreType.DMA,
        ] + [pltpu.VMEM((_CHUNK,), jnp.int32) for _ in range(nb)]
        + isems + tail_types,
    )
    def deg_kernel(dst_hbm, ones_hbm, zeros_hbm, out_hbm, ones_v,
                   acc_sh, ssem, *args):
        idx = args[:nb]
        isem = args[nb:2 * nb]
        c = lax.axis_index("c")
        s = lax.axis_index("s")
        base = (c * _NS + s) * epw

        pltpu.sync_copy(ones_hbm, ones_v)

        @pl.when(s == 0)
        def _():
            pltpu.sync_copy(zeros_hbm, acc_sh)

        plsc.subcore_barrier()

        def do_group(off, count):
            idescs = []
            for b in range(count):
                idescs.append(pltpu.async_copy(
                    dst_hbm.at[pl.ds(off + b * _CHUNK, _CHUNK)],
                    idx[b], isem[b]))
            sdescs = []
            for b in range(count):
                idescs[b].wait()
                sdescs.append(pltpu.async_copy(
                    ones_v, acc_sh.at[idx[b]], ssem, add=True))
            for d in sdescs:
                d.wait()

        def body(g, carry):
            do_group(base + g * (nb * _CHUNK), nb)
            return carry

        lax.fori_loop(0, ngroup, body, 0)
        if gtail:
            do_group(base + ngroup * (nb * _CHUNK), gtail)
        if etail:
            idx_t, ones_t = args[2 * nb:]
            pltpu.sync_copy(ones_hbm.at[pl.ds(0, etail)], ones_t)
            pltpu.sync_copy(dst_hbm.at[pl.ds(base + nchunk * _CHUNK, etail)],
                            idx_t)
            pltpu.sync_copy(ones_t, acc_sh.at[idx_t], add=True)
        plsc.subcore_barrier()

        @pl.when(s == 0)
        def _():
            pltpu.sync_copy(acc_sh, out_hbm.at[c])

    return deg_kernel(dst, ones_c, zeros_n)


def _sc_agg(h, src, dst, zeros_nd):
    n, d = h.shape
    e = src.shape[0]
    epw = e // _NW
    nchunk = epw // _CHUNK_A
    ngroup = nchunk // _NBA
    assert nchunk % _NBA == 0
    etail = epw - nchunk * _CHUNK_A

    sem_types = [pltpu.SemaphoreType.DMA for _ in range(3 * _NBA)]
    idx_types = [pltpu.VMEM((_CHUNK_A,), jnp.int32) for _ in range(2 * _NBA)]
    row_types = [pltpu.VMEM((_CHUNK_A, d), jnp.float32) for _ in range(_NBA)]
    tail_types = ([pltpu.VMEM((etail,), jnp.int32),
                   pltpu.VMEM((etail,), jnp.int32),
                   pltpu.VMEM((etail, d), jnp.float32)] if etail else [])

    @functools.partial(
        pl.kernel,
        out_type=jax.ShapeDtypeStruct((_NC, n, d), jnp.float32),
        mesh=_sc_mesh(),
        scratch_types=[pltpu.VMEM_SHARED((n, d), jnp.float32)]
        + idx_types + row_types + sem_types + tail_types,
    )
    def agg_kernel(h_hbm, src_hbm, dst_hbm, zeros_hbm, out_hbm,
                   acc_sh, *args):
        sidx = args[:_NBA]
        didx = args[_NBA:2 * _NBA]
        rows = args[2 * _NBA:3 * _NBA]
        isem = args[3 * _NBA:4 * _NBA]
        gsem = args[4 * _NBA:5 * _NBA]
        ssem = args[5 * _NBA:6 * _NBA]
        c = lax.axis_index("c")
        s = lax.axis_index("s")
        base = (c * _NS + s) * epw

        ra = (n // _NS + 7) // 8 * 8
        for k in range(_NS):
            r0 = k * ra
            rk = min(ra, n - r0)
            if rk > 0:
                @pl.when(s == k)
                def _(r0=r0, rk=rk):
                    pltpu.sync_copy(zeros_hbm.at[pl.ds(0, rk)],
                                    acc_sh.at[pl.ds(r0, rk)])

        plsc.subcore_barrier()

        for b in range(_NBA):
            cb = base + b * _CHUNK_A
            pltpu.sync_copy(src_hbm.at[pl.ds(cb, _CHUNK_A)], sidx[b])
            pltpu.sync_copy(dst_hbm.at[pl.ds(cb, _CHUNK_A)], didx[b])
            pltpu.async_copy(h_hbm.at[sidx[b]], rows[b], gsem[b])

        def body(g, carry):
            sdescs = []
            for b in range(_NBA):
                pltpu.make_async_copy(
                    h_hbm.at[sidx[b]], rows[b], gsem[b]).wait()
                sdescs.append(pltpu.async_copy(
                    rows[b], acc_sh.at[didx[b]], ssem[b], add=True))
            idescs = []
            for b in range(_NBA):
                ch2 = _NBA * g + b + _NBA
                sdescs[b].wait()

                @pl.when(ch2 < nchunk)
                def _(ch2=ch2, b=b):
                    cb = base + ch2 * _CHUNK_A
                    pltpu.async_copy(src_hbm.at[pl.ds(cb, _CHUNK_A)],
                                     sidx[b], isem[b])
                    pltpu.async_copy(dst_hbm.at[pl.ds(cb, _CHUNK_A)],
                                     didx[b], isem[b])
            for b in range(_NBA):
                ch2 = _NBA * g + b + _NBA

                @pl.when(ch2 < nchunk)
                def _(ch2=ch2, b=b):
                    cb = base + ch2 * _CHUNK_A
                    pltpu.make_async_copy(
                        src_hbm.at[pl.ds(cb, _CHUNK_A)], sidx[b],
                        isem[b]).wait()
                    pltpu.make_async_copy(
                        dst_hbm.at[pl.ds(cb, _CHUNK_A)], didx[b],
                        isem[b]).wait()
                    pltpu.async_copy(h_hbm.at[sidx[b]], rows[b], gsem[b])
            return carry

        lax.fori_loop(0, ngroup, body, 0)
        if etail:
            sidx_t, didx_t, rows_t = args[6 * _NBA:]
            toff = base + nchunk * _CHUNK_A
            pltpu.sync_copy(src_hbm.at[pl.ds(toff, etail)], sidx_t)
            pltpu.sync_copy(dst_hbm.at[pl.ds(toff, etail)], didx_t)
            pltpu.async_copy(h_hbm.at[sidx_t], rows_t, gsem[0]).wait()
            pltpu.sync_copy(rows_t, acc_sh.at[didx_t], add=True)
        plsc.subcore_barrier()

        for k in range(_NS):
            r0 = k * ra
            rk = min(ra, n - r0)
            if rk > 0:
                @pl.when(s == k)
                def _(r0=r0, rk=rk):
                    pltpu.sync_copy(acc_sh.at[pl.ds(r0, rk)],
                                    out_hbm.at[c, pl.ds(r0, rk)])

    return agg_kernel(h, src, dst, zeros_nd)


def _dis_from_partials(degp):
    deg = degp[0] + degp[1]
    return jnp.where(deg > 0, lax.rsqrt(deg), 0.0)


def _tc1_body(x_ref, w1_ref, degp_ref, out_ref):
    dis = _dis_from_partials(degp_ref[...])
    h = lax.dot_general(x_ref[...], w1_ref[...], (((1,), (1,)), ((), ())),
                        preferred_element_type=jnp.float32)
    out_ref[...] = h * dis[:, None]


def _tc2_body(aggp_ref, degp_ref, w2_ref, s1_ref, h2s_ref):
    dis = _dis_from_partials(degp_ref[...])
    agg = (aggp_ref[0] + aggp_ref[1]) * dis[:, None]
    mean = jnp.mean(agg, axis=0, keepdims=True)
    var = jnp.mean((agg - mean) ** 2, axis=0, keepdims=True)
    s1 = jnp.maximum((agg - mean) / jnp.sqrt(var + _EPS), 0.0)
    s1_ref[...] = s1
    h2 = lax.dot_general(s1, w2_ref[...], (((1,), (1,)), ((), ())),
                         preferred_element_type=jnp.float32)
    h2s_ref[...] = h2 * dis[:, None]


def _tc3_body(aggp_ref, degp_ref, s1_ref, w3_ref, out_ref):
    dis = _dis_from_partials(degp_ref[...])
    agg = (aggp_ref[0] + aggp_ref[1]) * dis[:, None]
    mean = jnp.mean(agg, axis=0, keepdims=True)
    var = jnp.mean((agg - mean) ** 2, axis=0, keepdims=True)
    s2 = jnp.maximum((agg - mean) / jnp.sqrt(var + _EPS), 0.0)
    d = s1_ref.shape[1]
    w3a = w3_ref[:, :d]
    w3b = w3_ref[:, d:]
    out_ref[...] = (
        lax.dot_general(s1_ref[...], w3a, (((1,), (1,)), ((), ())),
                        preferred_element_type=jnp.float32)
        + lax.dot_general(s2, w3b, (((1,), (1,)), ((), ())),
                          preferred_element_type=jnp.float32)
    )


def kernel(x, edge_index, W1, W2, W3):
    n, d_in = x.shape
    d_h = W1.shape[0]
    d_out = W3.shape[0]
    src = edge_index[0]
    dst = edge_index[1]

    ones_c = jnp.ones((_CHUNK,), jnp.float32)
    zeros_n = jnp.zeros((n,), jnp.float32)
    zeros_nd = jnp.zeros((n, d_h), jnp.float32)

    degp = _sc_deg(dst, ones_c, zeros_n)

    h1s = pl.pallas_call(
        _tc1_body,
        out_shape=jax.ShapeDtypeStruct((n, d_h), jnp.float32),
    )(x, W1, degp)

    agg1p = _sc_agg(h1s, src, dst, zeros_nd)

    s1, h2s = pl.pallas_call(
        _tc2_body,
        out_shape=[
            jax.ShapeDtypeStruct((n, d_h), jnp.float32),
            jax.ShapeDtypeStruct((n, d_h), jnp.float32),
        ],
    )(agg1p, degp, W2)

    agg2p = _sc_agg(h2s, src, dst, zeros_nd)

    out = pl.pallas_call(
        _tc3_body,
        out_shape=jax.ShapeDtypeStruct((n, d_out), jnp.float32),
    )(agg2p, degp, s1, W3)

    return out


# speedup vs baseline: 1.1233x; 1.1233x over previous
import functools

import jax
import jax.numpy as jnp
from jax import lax
from jax.experimental import pallas as pl
from jax.experimental.pallas import tpu as pltpu
from jax.experimental.pallas import tpu_sc as plsc

_EPS = 1e-5

_NC = 2
_NS = 16
_NW = _NC * _NS

_CHUNK = 128

_CHUNK_A = 64
_NBA = 4


def _sc_mesh():
    return plsc.VectorSubcoreMesh(
        core_axis_name="c", subcore_axis_name="s",
        num_cores=_NC, num_subcores=_NS)


def _sc_deg(dst, ones_c, zeros_n):
    n = zeros_n.shape[0]
    e = dst.shape[0]
    epw = e // _NW
    nchunk = epw // _CHUNK
    nb = 6
    ngroup = nchunk // nb
    gtail = nchunk - ngroup * nb
    etail = epw - nchunk * _CHUNK

    isems = [pltpu.SemaphoreType.DMA for _ in range(nb)]
    tail_types = ([pltpu.VMEM((etail,), jnp.int32),
                   pltpu.VMEM((etail,), jnp.float32)] if etail else [])

    @functools.partial(
        pl.kernel,
        out_type=jax.ShapeDtypeStruct((_NC, n), jnp.float32),
        mesh=_sc_mesh(),
        scratch_types=[
            pltpu.VMEM((_CHUNK,), jnp.float32),
            pltpu.VMEM_SHARED((n,), jnp.float32),
            pltpu.SemaphoreType.DMA,
        ] + [pltpu.VMEM((_CHUNK,), jnp.int32) for _ in range(nb)]
        + isems + tail_types,
    )
    def deg_kernel(dst_hbm, ones_hbm, zeros_hbm, out_hbm, ones_v,
                   acc_sh, ssem, *args):
        idx = args[:nb]
        isem = args[nb:2 * nb]
        c = lax.axis_index("c")
        s = lax.axis_index("s")
        base = (c * _NS + s) * epw

        pltpu.sync_copy(ones_hbm, ones_v)

        @pl.when(s == 0)
        def _():
            pltpu.sync_copy(zeros_hbm, acc_sh)

        plsc.subcore_barrier()

        def do_group(off, count):
            idescs = []
            for b in range(count):
                idescs.append(pltpu.async_copy(
                    dst_hbm.at[pl.ds(off + b * _CHUNK, _CHUNK)],
                    idx[b], isem[b]))
            sdescs = []
            for b in range(count):
                idescs[b].wait()
                sdescs.append(pltpu.async_copy(
                    ones_v, acc_sh.at[idx[b]], ssem, add=True))
            for d in sdescs:
                d.wait()

        def body(g, carry):
            do_group(base + g * (nb * _CHUNK), nb)
            return carry

        lax.fori_loop(0, ngroup, body, 0)
        if gtail:
            do_group(base + ngroup * (nb * _CHUNK), gtail)
        if etail:
            idx_t, ones_t = args[2 * nb:]
            pltpu.sync_copy(ones_hbm.at[pl.ds(0, etail)], ones_t)
            pltpu.sync_copy(dst_hbm.at[pl.ds(base + nchunk * _CHUNK, etail)],
                            idx_t)
            pltpu.sync_copy(ones_t, acc_sh.at[idx_t], add=True)
        plsc.subcore_barrier()

        @pl.when(s == 0)
        def _():
            pltpu.sync_copy(acc_sh, out_hbm.at[c])

    return deg_kernel(dst, ones_c, zeros_n)


def _sc_agg(h, src, dst, zeros_nd):
    n, d = h.shape
    e = src.shape[0]
    epw = e // _NW
    nchunk = epw // _CHUNK_A
    ngroup = nchunk // _NBA
    assert nchunk % _NBA == 0
    etail = epw - nchunk * _CHUNK_A

    sem_types = [pltpu.SemaphoreType.DMA for _ in range(3 * _NBA)]
    idx_types = [pltpu.VMEM((_CHUNK_A,), jnp.int32) for _ in range(2 * _NBA)]
    row_types = [pltpu.VMEM((_CHUNK_A, d), jnp.float32) for _ in range(_NBA)]
    tail_types = ([pltpu.VMEM((etail,), jnp.int32),
                   pltpu.VMEM((etail,), jnp.int32),
                   pltpu.VMEM((etail, d), jnp.float32)] if etail else [])

    @functools.partial(
        pl.kernel,
        out_type=jax.ShapeDtypeStruct((_NC, n, d), jnp.float32),
        mesh=_sc_mesh(),
        scratch_types=[pltpu.VMEM_SHARED((n, d), jnp.float32)]
        + idx_types + row_types + sem_types + tail_types,
    )
    def agg_kernel(h_hbm, src_hbm, dst_hbm, zeros_hbm, out_hbm,
                   acc_sh, *args):
        sidx = args[:_NBA]
        didx = args[_NBA:2 * _NBA]
        rows = args[2 * _NBA:3 * _NBA]
        isem = args[3 * _NBA:4 * _NBA]
        gsem = args[4 * _NBA:5 * _NBA]
        ssem = args[5 * _NBA:6 * _NBA]
        c = lax.axis_index("c")
        s = lax.axis_index("s")
        base = (c * _NS + s) * epw

        ra = (n // _NS + 7) // 8 * 8
        for k in range(_NS):
            r0 = k * ra
            rk = min(ra, n - r0)
            if rk > 0:
                @pl.when(s == k)
                def _(r0=r0, rk=rk):
                    pltpu.sync_copy(zeros_hbm.at[pl.ds(0, rk)],
                                    acc_sh.at[pl.ds(r0, rk)])

        plsc.subcore_barrier()

        pdescs = []
        for b in range(_NBA):
            cb = base + b * _CHUNK_A
            pdescs.append((
                pltpu.async_copy(src_hbm.at[pl.ds(cb, _CHUNK_A)],
                                 sidx[b], isem[b]),
                pltpu.async_copy(dst_hbm.at[pl.ds(cb, _CHUNK_A)],
                                 didx[b], isem[b]),
            ))
        for b in range(_NBA):
            pdescs[b][0].wait()
            pdescs[b][1].wait()
            pltpu.async_copy(h_hbm.at[sidx[b]], rows[b], gsem[b])

        def body(g, carry):
            sdescs = []
            for b in range(_NBA):
                pltpu.make_async_copy(
                    h_hbm.at[sidx[b]], rows[b], gsem[b]).wait()
                sdescs.append(pltpu.async_copy(
                    rows[b], acc_sh.at[didx[b]], ssem[b], add=True))
            idescs = []
            for b in range(_NBA):
                ch2 = _NBA * g + b + _NBA
                sdescs[b].wait()

                @pl.when(ch2 < nchunk)
                def _(ch2=ch2, b=b):
                    cb = base + ch2 * _CHUNK_A
                    pltpu.async_copy(src_hbm.at[pl.ds(cb, _CHUNK_A)],
                                     sidx[b], isem[b])
                    pltpu.async_copy(dst_hbm.at[pl.ds(cb, _CHUNK_A)],
                                     didx[b], isem[b])
            for b in range(_NBA):
                ch2 = _NBA * g + b + _NBA

                @pl.when(ch2 < nchunk)
                def _(ch2=ch2, b=b):
                    cb = base + ch2 * _CHUNK_A
                    pltpu.make_async_copy(
                        src_hbm.at[pl.ds(cb, _CHUNK_A)], sidx[b],
                        isem[b]).wait()
                    pltpu.make_async_copy(
                        dst_hbm.at[pl.ds(cb, _CHUNK_A)], didx[b],
                        isem[b]).wait()
                    pltpu.async_copy(h_hbm.at[sidx[b]], rows[b], gsem[b])
            return carry

        lax.fori_loop(0, ngroup, body, 0)
        if etail:
            sidx_t, didx_t, rows_t = args[6 * _NBA:]
            toff = base + nchunk * _CHUNK_A
            pltpu.sync_copy(src_hbm.at[pl.ds(toff, etail)], sidx_t)
            pltpu.sync_copy(dst_hbm.at[pl.ds(toff, etail)], didx_t)
            pltpu.async_copy(h_hbm.at[sidx_t], rows_t, gsem[0]).wait()
            pltpu.sync_copy(rows_t, acc_sh.at[didx_t], add=True)
        plsc.subcore_barrier()

        for k in range(_NS):
            r0 = k * ra
            rk = min(ra, n - r0)
            if rk > 0:
                @pl.when(s == k)
                def _(r0=r0, rk=rk):
                    pltpu.sync_copy(acc_sh.at[pl.ds(r0, rk)],
                                    out_hbm.at[c, pl.ds(r0, rk)])

    return agg_kernel(h, src, dst, zeros_nd)


def _dis_from_partials(degp):
    deg = degp[0] + degp[1]
    return jnp.where(deg > 0, lax.rsqrt(deg), 0.0)


def _tc1_body(x_ref, w1_ref, degp_ref, out_ref):
    dis = _dis_from_partials(degp_ref[...])
    h = lax.dot_general(x_ref[...], w1_ref[...], (((1,), (1,)), ((), ())),
                        preferred_element_type=jnp.float32)
    out_ref[...] = h * dis[:, None]


def _tc2_body(aggp_ref, degp_ref, w2_ref, s1_ref, h2s_ref):
    dis = _dis_from_partials(degp_ref[...])
    agg = (aggp_ref[0] + aggp_ref[1]) * dis[:, None]
    mean = jnp.mean(agg, axis=0, keepdims=True)
    var = jnp.mean((agg - mean) ** 2, axis=0, keepdims=True)
    s1 = jnp.maximum((agg - mean) / jnp.sqrt(var + _EPS), 0.0)
    s1_ref[...] = s1
    h2 = lax.dot_general(s1, w2_ref[...], (((1,), (1,)), ((), ())),
                         preferred_element_type=jnp.float32)
    h2s_ref[...] = h2 * dis[:, None]


def _tc3_body(aggp_ref, degp_ref, s1_ref, w3_ref, out_ref):
    dis = _dis_from_partials(degp_ref[...])
    agg = (aggp_ref[0] + aggp_ref[1]) * dis[:, None]
    mean = jnp.mean(agg, axis=0, keepdims=True)
    var = jnp.mean((agg - mean) ** 2, axis=0, keepdims=True)
    s2 = jnp.maximum((agg - mean) / jnp.sqrt(var + _EPS), 0.0)
    d = s1_ref.shape[1]
    w3a = w3_ref[:, :d]
    w3b = w3_ref[:, d:]
    out_ref[...] = (
        lax.dot_general(s1_ref[...], w3a, (((1,), (1,)), ((), ())),
                        preferred_element_type=jnp.float32)
        + lax.dot_general(s2, w3b, (((1,), (1,)), ((), ())),
                          preferred_element_type=jnp.float32)
    )


def kernel(x, edge_index, W1, W2, W3):
    n, d_in = x.shape
    d_h = W1.shape[0]
    d_out = W3.shape[0]
    src = edge_index[0]
    dst = edge_index[1]

    ones_c = jnp.ones((_CHUNK,), jnp.float32)
    zeros_n = jnp.zeros((n,), jnp.float32)
    zeros_nd = jnp.zeros((n, d_h), jnp.float32)

    degp = _sc_deg(dst, ones_c, zeros_n)

    h1s = pl.pallas_call(
        _tc1_body,
        out_shape=jax.ShapeDtypeStruct((n, d_h), jnp.float32),
    )(x, W1, degp)

    agg1p = _sc_agg(h1s, src, dst, zeros_nd)

    s1, h2s = pl.pallas_call(
        _tc2_body,
        out_shape=[
            jax.ShapeDtypeStruct((n, d_h), jnp.float32),
            jax.ShapeDtypeStruct((n, d_h), jnp.float32),
        ],
    )(agg1p, degp, W2)

    agg2p = _sc_agg(h2s, src, dst, zeros_nd)

    out = pl.pallas_call(
        _tc3_body,
        out_shape=jax.ShapeDtypeStruct((n, d_out), jnp.float32),
    )(agg2p, degp, s1, W3)

    return out
